# Initial kernel scaffold; baseline (speedup 1.0000x reference)
import jax
import jax.numpy as jnp
from jax.experimental import pallas as pl


def kernel(features, edge_index, W0, W1):
    raise NotImplementedError("write your pallas kernel here")


# speedup vs baseline: 8.1472x; 8.1472x over previous
import functools

import jax
import jax.numpy as jnp
from jax import lax
from jax.experimental import pallas as pl
from jax.experimental.pallas import tpu as pltpu
from jax.experimental.pallas import tpu_sc as plsc

N_NODES = 10000
IN_DIM = 128
HID = 16
NCLS = 47

NT = 32
CHUNK = 128
K_CH = 80
EPT = CHUNK * K_CH
E_PAD = NT * EPT
ROWS_PT = 632
N_ACC = ROWS_PT * 16


def _mm_relu(x, w):
    def body(x_ref, w_ref, o_ref):
        o_ref[...] = jnp.maximum(
            jnp.dot(x_ref[...], w_ref[...], preferred_element_type=jnp.float32),
            0.0)
    return pl.pallas_call(
        body,
        out_shape=jax.ShapeDtypeStruct((x.shape[0], w.shape[1]), jnp.float32),
    )(x, w)


def _sum2(p):
    def body(p_ref, o_ref):
        o_ref[...] = p_ref[:N_ACC] + p_ref[N_ACC:]
    return pl.pallas_call(
        body,
        out_shape=jax.ShapeDtypeStruct((N_ACC, HID), jnp.float32),
    )(p)


def _sum_mm(p, w):
    def body(p_ref, w_ref, o_ref):
        h = p_ref[:N_ACC] + p_ref[N_ACC:]
        r = jnp.dot(h, w_ref[...], preferred_element_type=jnp.float32)
        o_ref[...] = r[:N_NODES]
    return pl.pallas_call(
        body,
        out_shape=jax.ShapeDtypeStruct((N_NODES, NCLS), jnp.float32),
    )(p, w)


def _spmm_partial(h, src_r, dst_r, zeros):
    mesh = plsc.VectorSubcoreMesh(core_axis_name="c", subcore_axis_name="s")

    @functools.partial(
        pl.kernel,
        out_type=jax.ShapeDtypeStruct((2 * N_ACC, HID), jnp.float32),
        mesh=mesh,
        scratch_types=[
            pltpu.VMEM_SHARED((N_ACC, HID), jnp.float32),
            pltpu.VMEM((CHUNK,), jnp.int32),
            pltpu.VMEM((CHUNK,), jnp.int32),
            pltpu.VMEM((CHUNK, HID), jnp.float32),
            pltpu.SemaphoreType.DMA,
        ],
        compiler_params=pltpu.CompilerParams(use_tc_tiling_on_sc=False),
    )
    def spmm(h_hbm, src_hbm, dst_hbm, z_hbm, out_hbm, acc, sidx, didx, rows,
             gsem):
        c = lax.axis_index("c")
        s = lax.axis_index("s")
        w = s * 2 + c

        pltpu.sync_copy(z_hbm.at[pl.ds(s * ROWS_PT, ROWS_PT)],
                        acc.at[pl.ds(s * ROWS_PT, ROWS_PT)])
        plsc.subcore_barrier()

        def chunk(j, carry):
            base = w * K_CH + j
            pltpu.sync_copy(src_hbm.at[base], sidx)
            pltpu.sync_copy(dst_hbm.at[base], didx)
            pltpu.async_copy(h_hbm.at[sidx], rows, gsem).wait()
            pltpu.sync_copy(rows, acc.at[didx], add=True)
            return carry
        lax.fori_loop(0, K_CH, chunk, 0)
        plsc.subcore_barrier()

        pltpu.sync_copy(
            acc.at[pl.ds(s * ROWS_PT, ROWS_PT)],
            out_hbm.at[pl.ds(c * N_ACC + s * ROWS_PT, ROWS_PT)])

    return spmm(h, src_r, dst_r, zeros)


def kernel(features, edge_index, W0, W1):
    src = edge_index[0].astype(jnp.int32)
    dst = edge_index[1].astype(jnp.int32)
    pad = E_PAD - src.shape[0]
    src = jnp.concatenate([src, jnp.zeros((pad,), jnp.int32)])
    dst = jnp.concatenate([dst, jnp.full((pad,), N_NODES, jnp.int32)])
    src_r = src.reshape(NT * K_CH, CHUNK)
    dst_r = dst.reshape(NT * K_CH, CHUNK)
    zeros = jnp.zeros((N_ACC, HID), jnp.float32)

    h = _mm_relu(features, W0)
    p1 = _spmm_partial(h, src_r, dst_r, zeros)
    h2 = _sum2(p1)
    p2 = _spmm_partial(h2, src_r, dst_r, zeros)
    return _sum_mm(p2, W1)


# speedup vs baseline: 16.1343x; 1.9804x over previous
import functools

import jax
import jax.numpy as jnp
from jax import lax
from jax.experimental import pallas as pl
from jax.experimental.pallas import tpu as pltpu
from jax.experimental.pallas import tpu_sc as plsc

N_NODES = 10000
IN_DIM = 128
HID = 16
NCLS = 47

NT = 32
CHUNK = 128
K_CH = 80
G = 4
NG = K_CH // G
EPT = CHUNK * K_CH
E_PAD = NT * EPT
ROWS_PT = 632
N_ACC = ROWS_PT * 16


def _mm_relu(x, w):
    def body(x_ref, w_ref, o_ref):
        o_ref[...] = jnp.maximum(
            jnp.dot(x_ref[...], w_ref[...], preferred_element_type=jnp.float32),
            0.0)
    return pl.pallas_call(
        body,
        out_shape=jax.ShapeDtypeStruct((x.shape[0], w.shape[1]), jnp.float32),
    )(x, w)


def _sum2(p):
    def body(p_ref, o_ref):
        o_ref[...] = p_ref[:N_ACC] + p_ref[N_ACC:]
    return pl.pallas_call(
        body,
        out_shape=jax.ShapeDtypeStruct((N_ACC, HID), jnp.float32),
    )(p)


def _sum_mm(p, w):
    def body(p_ref, w_ref, o_ref):
        h = p_ref[:N_ACC] + p_ref[N_ACC:]
        r = jnp.dot(h, w_ref[...], preferred_element_type=jnp.float32)
        o_ref[...] = r[:N_NODES]
    return pl.pallas_call(
        body,
        out_shape=jax.ShapeDtypeStruct((N_NODES, NCLS), jnp.float32),
    )(p, w)


def _spmm_partial(h, src_r, dst_r, zeros):
    mesh = plsc.VectorSubcoreMesh(core_axis_name="c", subcore_axis_name="s")

    @functools.partial(
        pl.kernel,
        out_type=jax.ShapeDtypeStruct((2 * N_ACC, HID), jnp.float32),
        mesh=mesh,
        scratch_types=[
            pltpu.VMEM_SHARED((N_ACC, HID), jnp.float32),
            pltpu.VMEM((K_CH, CHUNK), jnp.int32),
            pltpu.VMEM((K_CH, CHUNK), jnp.int32),
            pltpu.VMEM((G * CHUNK, HID), jnp.float32),
            pltpu.VMEM((G * CHUNK, HID), jnp.float32),
            pltpu.SemaphoreType.DMA,
            pltpu.SemaphoreType.DMA,
            pltpu.SemaphoreType.DMA,
            pltpu.SemaphoreType.DMA,
        ],
        compiler_params=pltpu.CompilerParams(use_tc_tiling_on_sc=False),
    )
    def spmm(h_hbm, src_hbm, dst_hbm, z_hbm, out_hbm, acc, sidx, didx,
             buf_a, buf_b, gsem_a, gsem_b, ssem_a, ssem_b):
        c = lax.axis_index("c")
        s = lax.axis_index("s")
        w = s * 2 + c

        pltpu.sync_copy(z_hbm.at[pl.ds(s * ROWS_PT, ROWS_PT)],
                        acc.at[pl.ds(s * ROWS_PT, ROWS_PT)])
        pltpu.sync_copy(src_hbm.at[pl.ds(w * K_CH, K_CH)], sidx)
        pltpu.sync_copy(dst_hbm.at[pl.ds(w * K_CH, K_CH)], didx)
        plsc.subcore_barrier()

        def issue_gathers(g, buf, sem):
            for i in range(G):
                pltpu.async_copy(h_hbm.at[sidx.at[g * G + i]],
                                 buf.at[pl.ds(i * CHUNK, CHUNK)], sem)

        def issue_scatters(g, buf, sem):
            for i in range(G):
                pltpu.async_copy(buf.at[pl.ds(i * CHUNK, CHUNK)],
                                 acc.at[didx.at[g * G + i]], sem, add=True)

        def drain(buf, sem):
            pltpu.make_async_copy(z_hbm.at[pl.ds(0, G * CHUNK)], buf,
                                  sem).wait()

        issue_gathers(0, buf_a, gsem_a)
        issue_gathers(1, buf_b, gsem_b)
        drain(buf_a, gsem_a)
        issue_scatters(0, buf_a, ssem_a)

        def body(gg, carry):
            h1 = 2 * gg + 1
            drain(buf_a, ssem_a)
            issue_gathers(h1 + 1, buf_a, gsem_a)
            drain(buf_b, gsem_b)
            issue_scatters(h1, buf_b, ssem_b)
            drain(buf_a, gsem_a)
            issue_scatters(h1 + 1, buf_a, ssem_a)
            drain(buf_b, ssem_b)
            issue_gathers(h1 + 2, buf_b, gsem_b)
            return carry
        lax.fori_loop(0, (NG - 2) // 2, body, 0)

        drain(buf_a, ssem_a)
        drain(buf_b, gsem_b)
        issue_scatters(NG - 1, buf_b, ssem_b)
        drain(buf_b, ssem_b)
        plsc.subcore_barrier()

        pltpu.sync_copy(
            acc.at[pl.ds(s * ROWS_PT, ROWS_PT)],
            out_hbm.at[pl.ds(c * N_ACC + s * ROWS_PT, ROWS_PT)])

    return spmm(h, src_r, dst_r, zeros)


def kernel(features, edge_index, W0, W1):
    src = edge_index[0].astype(jnp.int32)
    dst = edge_index[1].astype(jnp.int32)
    pad = E_PAD - src.shape[0]
    src = jnp.concatenate([src, jnp.zeros((pad,), jnp.int32)])
    dst = jnp.concatenate([dst, jnp.full((pad,), N_NODES, jnp.int32)])
    src_r = src.reshape(NT * K_CH, CHUNK)
    dst_r = dst.reshape(NT * K_CH, CHUNK)
    zeros = jnp.zeros((N_ACC, HID), jnp.float32)

    h = _mm_relu(features, W0)
    p1 = _spmm_partial(h, src_r, dst_r, zeros)
    h2 = _sum2(p1)
    p2 = _spmm_partial(h2, src_r, dst_r, zeros)
    return _sum_mm(p2, W1)


# speedup vs baseline: 23.7105x; 1.4696x over previous
import functools

import jax
import jax.numpy as jnp
from jax import lax
from jax.experimental import pallas as pl
from jax.experimental.pallas import tpu as pltpu
from jax.experimental.pallas import tpu_sc as plsc

N_NODES = 10000
IN_DIM = 128
HID = 16
NCLS = 47

NT = 32
CHUNK = 125
K_CH = 80
G = 4
NG = K_CH // G
ROWS_PT = 632
N_ACC = ROWS_PT * 16


def _mm_relu(x, w):
    def body(x_ref, w_ref, o_ref):
        o_ref[...] = jnp.maximum(
            jnp.dot(x_ref[...], w_ref[...], preferred_element_type=jnp.float32),
            0.0)
    return pl.pallas_call(
        body,
        out_shape=jax.ShapeDtypeStruct((x.shape[0], w.shape[1]), jnp.float32),
    )(x, w)


def _sum2(p):
    def body(p_ref, o_ref):
        o_ref[...] = p_ref[:N_ACC] + p_ref[N_ACC:]
    return pl.pallas_call(
        body,
        out_shape=jax.ShapeDtypeStruct((N_ACC, HID), jnp.float32),
    )(p)


def _sum_mm(p, w):
    def body(p_ref, w_ref, o_ref):
        h = p_ref[:N_ACC] + p_ref[N_ACC:]
        r = jnp.dot(h, w_ref[...], preferred_element_type=jnp.float32)
        o_ref[...] = r[:N_NODES]
    return pl.pallas_call(
        body,
        out_shape=jax.ShapeDtypeStruct((N_NODES, NCLS), jnp.float32),
    )(p, w)


def _spmm_partial(h, src_r, dst_r, zeros):
    mesh = plsc.VectorSubcoreMesh(core_axis_name="c", subcore_axis_name="s")

    @functools.partial(
        pl.kernel,
        out_type=jax.ShapeDtypeStruct((2 * N_ACC, HID), jnp.float32),
        mesh=mesh,
        scratch_types=[
            pltpu.VMEM_SHARED((N_ACC, HID), jnp.float32),
            pltpu.VMEM((K_CH, CHUNK), jnp.int32),
            pltpu.VMEM((K_CH, CHUNK), jnp.int32),
            pltpu.VMEM((G * CHUNK, HID), jnp.float32),
            pltpu.VMEM((G * CHUNK, HID), jnp.float32),
            pltpu.SemaphoreType.DMA,
            pltpu.SemaphoreType.DMA,
            pltpu.SemaphoreType.DMA,
            pltpu.SemaphoreType.DMA,
        ],
        compiler_params=pltpu.CompilerParams(use_tc_tiling_on_sc=False),
    )
    def spmm(h_hbm, src_hbm, dst_hbm, z_hbm, out_hbm, acc, sidx, didx,
             buf_a, buf_b, gsem_a, gsem_b, ssem_a, ssem_b):
        c = lax.axis_index("c")
        s = lax.axis_index("s")
        w = s * 2 + c

        pltpu.sync_copy(z_hbm.at[pl.ds(s * ROWS_PT, ROWS_PT)],
                        acc.at[pl.ds(s * ROWS_PT, ROWS_PT)])
        pltpu.sync_copy(src_hbm.at[pl.ds(w * K_CH, K_CH)], sidx)
        pltpu.sync_copy(dst_hbm.at[pl.ds(w * K_CH, K_CH)], didx)
        plsc.subcore_barrier()

        def issue_gathers(g, buf, sem):
            for i in range(G):
                pltpu.async_copy(h_hbm.at[sidx.at[g * G + i]],
                                 buf.at[pl.ds(i * CHUNK, CHUNK)], sem)

        def issue_scatters(g, buf, sem):
            for i in range(G):
                pltpu.async_copy(buf.at[pl.ds(i * CHUNK, CHUNK)],
                                 acc.at[didx.at[g * G + i]], sem, add=True)

        def drain(buf, sem):
            pltpu.make_async_copy(z_hbm.at[pl.ds(0, G * CHUNK)], buf,
                                  sem).wait()

        issue_gathers(0, buf_a, gsem_a)
        issue_gathers(1, buf_b, gsem_b)
        drain(buf_a, gsem_a)
        issue_scatters(0, buf_a, ssem_a)

        def body(gg, carry):
            h1 = 2 * gg + 1
            drain(buf_a, ssem_a)
            issue_gathers(h1 + 1, buf_a, gsem_a)
            drain(buf_b, gsem_b)
            issue_scatters(h1, buf_b, ssem_b)
            drain(buf_a, gsem_a)
            issue_scatters(h1 + 1, buf_a, ssem_a)
            drain(buf_b, ssem_b)
            issue_gathers(h1 + 2, buf_b, gsem_b)
            return carry
        lax.fori_loop(0, (NG - 2) // 2, body, 0)

        drain(buf_a, ssem_a)
        drain(buf_b, gsem_b)
        issue_scatters(NG - 1, buf_b, ssem_b)
        drain(buf_b, ssem_b)
        plsc.subcore_barrier()

        pltpu.sync_copy(
            acc.at[pl.ds(s * ROWS_PT, ROWS_PT)],
            out_hbm.at[pl.ds(c * N_ACC + s * ROWS_PT, ROWS_PT)])

    return spmm(h, src_r, dst_r, zeros)


def kernel(features, edge_index, W0, W1):
    src_r = edge_index[0].astype(jnp.int32).reshape(NT * K_CH, CHUNK)
    dst_r = edge_index[1].astype(jnp.int32).reshape(NT * K_CH, CHUNK)
    zeros = jnp.zeros((N_ACC, HID), jnp.float32)

    h = _mm_relu(features, W0)
    p1 = _spmm_partial(h, src_r, dst_r, zeros)
    h2 = _sum2(p1)
    p2 = _spmm_partial(h2, src_r, dst_r, zeros)
    return _sum_mm(p2, W1)


# speedup vs baseline: 25.1401x; 1.0603x over previous
import functools

import jax
import jax.numpy as jnp
from jax import lax
from jax.experimental import pallas as pl
from jax.experimental.pallas import tpu as pltpu
from jax.experimental.pallas import tpu_sc as plsc

N_NODES = 10000
IN_DIM = 128
HID = 16
NCLS = 47

NT = 32
CHUNK = 125
K_CH = 80
G = 4
NG = K_CH // G
ROWS_PT = 632
N_ACC = ROWS_PT * 16


def _mm_relu(x, w):
    def body(x_ref, w_ref, o_ref):
        o_ref[...] = jnp.maximum(
            jnp.dot(x_ref[...], w_ref[...], preferred_element_type=jnp.float32),
            0.0)
    return pl.pallas_call(
        body,
        out_shape=jax.ShapeDtypeStruct((x.shape[0], w.shape[1]), jnp.float32),
    )(x, w)


def _sum2(p):
    def body(p_ref, o_ref):
        o_ref[...] = p_ref[:N_ACC] + p_ref[N_ACC:]
    return pl.pallas_call(
        body,
        out_shape=jax.ShapeDtypeStruct((N_ACC, HID), jnp.float32),
    )(p)


def _sum_mm(p, w):
    def body(p_ref, w_ref, o_ref):
        h = p_ref[:N_ACC] + p_ref[N_ACC:]
        r = jnp.dot(h, w_ref[...], preferred_element_type=jnp.float32)
        o_ref[...] = r[:N_NODES]
    return pl.pallas_call(
        body,
        out_shape=jax.ShapeDtypeStruct((N_NODES, NCLS), jnp.float32),
    )(p, w)


def _spmm_partial(h, src_r, dst_r, zeros):
    mesh = plsc.VectorSubcoreMesh(core_axis_name="c", subcore_axis_name="s")
    n_h = h.shape[0]
    rpt_h = n_h // 16

    @functools.partial(
        pl.kernel,
        out_type=jax.ShapeDtypeStruct((2 * N_ACC, HID), jnp.float32),
        mesh=mesh,
        scratch_types=[
            pltpu.VMEM_SHARED((n_h, HID), jnp.float32),
            pltpu.VMEM_SHARED((N_ACC, HID), jnp.float32),
            pltpu.VMEM((K_CH, CHUNK), jnp.int32),
            pltpu.VMEM((K_CH, CHUNK), jnp.int32),
            pltpu.VMEM((G * CHUNK, HID), jnp.float32),
            pltpu.VMEM((G * CHUNK, HID), jnp.float32),
            pltpu.SemaphoreType.DMA,
            pltpu.SemaphoreType.DMA,
            pltpu.SemaphoreType.DMA,
            pltpu.SemaphoreType.DMA,
        ],
        compiler_params=pltpu.CompilerParams(use_tc_tiling_on_sc=False),
    )
    def spmm(h_hbm, src_hbm, dst_hbm, z_hbm, out_hbm, hsh, acc, sidx, didx,
             buf_a, buf_b, gsem_a, gsem_b, ssem_a, ssem_b):
        c = lax.axis_index("c")
        s = lax.axis_index("s")
        w = s * 2 + c

        pltpu.sync_copy(z_hbm.at[pl.ds(s * ROWS_PT, ROWS_PT)],
                        acc.at[pl.ds(s * ROWS_PT, ROWS_PT)])
        pltpu.sync_copy(h_hbm.at[pl.ds(s * rpt_h, rpt_h)],
                        hsh.at[pl.ds(s * rpt_h, rpt_h)])
        pltpu.sync_copy(src_hbm.at[pl.ds(w * K_CH, K_CH)], sidx)
        pltpu.sync_copy(dst_hbm.at[pl.ds(w * K_CH, K_CH)], didx)
        plsc.subcore_barrier()

        def issue_gathers(g, buf, sem):
            for i in range(G):
                pltpu.async_copy(hsh.at[sidx.at[g * G + i]],
                                 buf.at[pl.ds(i * CHUNK, CHUNK)], sem)

        def issue_scatters(g, buf, sem):
            for i in range(G):
                pltpu.async_copy(buf.at[pl.ds(i * CHUNK, CHUNK)],
                                 acc.at[didx.at[g * G + i]], sem, add=True)

        def drain(buf, sem):
            pltpu.make_async_copy(z_hbm.at[pl.ds(0, G * CHUNK)], buf,
                                  sem).wait()

        issue_gathers(0, buf_a, gsem_a)
        issue_gathers(1, buf_b, gsem_b)
        drain(buf_a, gsem_a)
        issue_scatters(0, buf_a, ssem_a)

        def body(gg, carry):
            h1 = 2 * gg + 1
            drain(buf_a, ssem_a)
            issue_gathers(h1 + 1, buf_a, gsem_a)
            drain(buf_b, gsem_b)
            issue_scatters(h1, buf_b, ssem_b)
            drain(buf_a, gsem_a)
            issue_scatters(h1 + 1, buf_a, ssem_a)
            drain(buf_b, ssem_b)
            issue_gathers(h1 + 2, buf_b, gsem_b)
            return carry
        lax.fori_loop(0, (NG - 2) // 2, body, 0)

        drain(buf_a, ssem_a)
        drain(buf_b, gsem_b)
        issue_scatters(NG - 1, buf_b, ssem_b)
        drain(buf_b, ssem_b)
        plsc.subcore_barrier()

        pltpu.sync_copy(
            acc.at[pl.ds(s * ROWS_PT, ROWS_PT)],
            out_hbm.at[pl.ds(c * N_ACC + s * ROWS_PT, ROWS_PT)])

    return spmm(h, src_r, dst_r, zeros)


def kernel(features, edge_index, W0, W1):
    src_r = edge_index[0].astype(jnp.int32).reshape(NT * K_CH, CHUNK)
    dst_r = edge_index[1].astype(jnp.int32).reshape(NT * K_CH, CHUNK)
    zeros = jnp.zeros((N_ACC, HID), jnp.float32)

    h = _mm_relu(features, W0)
    p1 = _spmm_partial(h, src_r, dst_r, zeros)
    h2 = _sum2(p1)
    p2 = _spmm_partial(h2, src_r, dst_r, zeros)
    return _sum_mm(p2, W1)
